# baseline (speedup 1.0000x reference)
import functools

import jax
import jax.numpy as jnp
from jax import lax
from jax.experimental import pallas as pl
from jax.experimental.pallas import tpu as pltpu
from jax.experimental.pallas import tpu_sc as plsc

N_NODES = 10000
N_EDGES = 320000
D_FEAT = 128
HID_HEADS = 8
HID_DIM = 8
NUM_CLASSES = 16

NC = 2
NS = 16
NW = NC * NS
CH = 128
NCH = 80
EPT = NCH * CH
EPAD = EPT * NW
R = 10240
DUMMY = 10016

_ROWBLK = 2000


def _edge_kernel(H, C, WID, XLW, XRW):
    D = H * C
    mesh = plsc.VectorSubcoreMesh(
        core_axis_name="c", subcore_axis_name="s", num_cores=NC,
        num_subcores=NS)
    rows_per_tile = R // NS
    copies = rows_per_tile // CH

    @functools.partial(
        pl.kernel,
        out_type=jax.ShapeDtypeStruct((NC, R, WID), jnp.float32),
        mesh=mesh,
        scratch_types=[
            pltpu.VMEM((NCH, CH), jnp.int32),
            pltpu.VMEM((NCH, CH), jnp.int32),
            pltpu.VMEM((CH, XLW), jnp.float32),
            pltpu.VMEM((CH, XLW), jnp.float32),
            pltpu.VMEM((CH, XRW), jnp.float32),
            pltpu.VMEM((CH, XRW), jnp.float32),
            pltpu.VMEM((CH, WID), jnp.float32),
            pltpu.VMEM((CH, WID), jnp.float32),
            pltpu.VMEM((D, 16), jnp.float32),
            pltpu.VMEM_SHARED((R, WID), jnp.float32),
            pltpu.SemaphoreType.DMA,
            pltpu.SemaphoreType.DMA,
            pltpu.SemaphoreType.DMA,
            pltpu.SemaphoreType.DMA,
            pltpu.SemaphoreType.DMA,
            pltpu.SemaphoreType.DMA,
        ],
        compiler_params=pltpu.CompilerParams(
            needs_layout_passes=False, use_tc_tiling_on_sc=False),
    )
    def ek(src_hbm, dst_hbm, xl_hbm, xr_hbm, att_hbm, out_hbm,
           src_all, dst_all, xl0, xl1, xr0, xr1, w0, w1, att_v, acc_sh,
           gl0, gl1, gr0, gr1, sc0, sc1):
        xls = (xl0, xl1)
        xrs = (xr0, xr1)
        ws = (w0, w1)
        gls = (gl0, gl1)
        grs = (gr0, gr1)
        scs = (sc0, sc1)
        cid = lax.axis_index("c")
        sid = lax.axis_index("s")
        wid = sid * NC + cid
        iota = lax.iota(jnp.int32, 16)
        d_att = pltpu.async_copy(att_hbm, att_v, sc0)
        d_src = pltpu.async_copy(src_hbm.at[wid], src_all, gl0)
        d_dst = pltpu.async_copy(dst_hbm.at[wid], dst_all, gr0)

        for w_v in ws:
            offs = list(range(0, WID - 15, 16))
            if WID % 16:
                offs.append(WID - 16)

            @pl.loop(0, CH)
            def _zw(r, w_v=w_v):
                for off in offs:
                    w_v[r, pl.ds(off, 16)] = jnp.zeros((16,), jnp.float32)

        r0 = sid * rows_per_tile
        dz = [pltpu.async_copy(w0, acc_sh.at[pl.ds(r0 + k * CH, CH)], sc1)
              for k in range(copies)]
        d_att.wait()
        d_src.wait()
        d_dst.wait()
        for d in dz:
            d.wait()
        plsc.subcore_barrier()

        def issue_gather(i, b):
            pltpu.async_copy(xl_hbm.at[src_all.at[i]], xls[b], gls[b])
            pltpu.async_copy(xr_hbm.at[dst_all.at[i]], xrs[b], grs[b])

        def wait_gather(i, b):
            pltpu.make_async_copy(
                xl_hbm.at[src_all.at[i]], xls[b], gls[b]).wait()
            pltpu.make_async_copy(
                xr_hbm.at[dst_all.at[i]], xrs[b], grs[b]).wait()

        def issue_scatter(i, b):
            pltpu.async_copy(ws[b], acc_sh.at[dst_all.at[i]], scs[b],
                             add=True)

        def wait_scatter(i, b):
            pltpu.make_async_copy(
                ws[b], acc_sh.at[dst_all.at[i]], scs[b]).wait()

        def compute(b):
            xl_r, xr_r, w_v = xls[b], xrs[b], ws[b]
            for h in range(H):
                atts = [att_v[h * C + c, :] for c in range(C)]

                @pl.loop(0, CH // 16)
                def _grp(g, h=h, atts=atts):
                    row = g * 16 + iota
                    acc = jnp.zeros((16,), jnp.float32)
                    vls = []
                    for c in range(C):
                        d = h * C + c
                        colv = jnp.full((16,), d, jnp.int32)
                        vl = plsc.load_gather(xl_r, [row, colv])
                        vr = plsc.load_gather(xr_r, [row, colv])
                        z = vl + vr
                        m = jnp.maximum(z, 0.2 * z)
                        acc = acc + m * atts[c]
                        vls.append(vl)
                    ee = jnp.exp(acc)
                    for c in range(C):
                        colv = jnp.full((16,), h * C + c, jnp.int32)
                        plsc.store_scatter(w_v, [row, colv], ee * vls[c])
                    plsc.store_scatter(
                        w_v, [row, jnp.full((16,), D + h, jnp.int32)], ee)

        issue_gather(0, 0)
        issue_gather(1, 1)
        for i in (0, 1):
            wait_gather(i, i)
            compute(i)
            issue_scatter(i, i)
            issue_gather(i + 2, i)

        @pl.loop(1, NCH // 2 - 1)
        def _j(j):
            for b in range(2):
                i = 2 * j + b
                wait_gather(i, b)
                wait_scatter(i - 2, b)
                compute(b)
                issue_scatter(i, b)
                issue_gather(i + 2, b)

        for b in range(2):
            i = NCH - 2 + b
            wait_gather(i, b)
            wait_scatter(i - 2, b)
            compute(b)
            issue_scatter(i, b)
        for b in range(2):
            wait_scatter(NCH - 2 + b, b)

        plsc.subcore_barrier()
        douts = []
        for k in range(copies):
            if k >= 2:
                douts[k - 2].wait()
            pltpu.sync_copy(acc_sh.at[pl.ds(r0 + k * CH, CH)], ws[k % 2])
            douts.append(pltpu.async_copy(
                ws[k % 2], out_hbm.at[cid, pl.ds(r0 + k * CH, CH)],
                scs[k % 2]))
        douts[-2].wait()
        douts[-1].wait()

    return ek


def _mm1_body(x_ref, wl_ref, wr_ref, xl_ref, xr_ref):
    xx = x_ref[...]
    l = jnp.dot(xx, wl_ref[...], preferred_element_type=jnp.float32)
    b = xx.shape[0]
    xr_ref[...] = jnp.concatenate(
        [jnp.dot(xx, wr_ref[...], preferred_element_type=jnp.float32),
         jnp.zeros((b, 1), jnp.float32)], axis=1)
    xl_ref[...] = jnp.concatenate(
        [l, jnp.zeros((b, 1), jnp.float32)], axis=1)


def _mid_body(a0_ref, a1_ref, xl_ref, xr_ref, att_ref, b1_ref, wl_ref,
              wr_ref, xl2_ref, xr2_ref):
    s = a0_ref[...] + a1_ref[...]
    rep = (lax.broadcasted_iota(jnp.int32, (8, 64), 1) // 8
           == lax.broadcasted_iota(jnp.int32, (8, 64), 0)).astype(jnp.float32)
    rep_t = (lax.broadcasted_iota(jnp.int32, (64, 8), 0) // 8
             == lax.broadcasted_iota(jnp.int32, (64, 8), 1)).astype(jnp.float32)
    xl = xl_ref[:, :64]
    z = xl + xr_ref[:, :64]
    m = jnp.maximum(z, 0.2 * z) * att_ref[...]
    ee = jnp.exp(jnp.dot(m, rep_t, preferred_element_type=jnp.float32))
    num = s[:, :64] + jnp.dot(ee, rep,
                              preferred_element_type=jnp.float32) * xl
    den = s[:, 64:72] + ee
    den_rep = jnp.dot(den, rep, preferred_element_type=jnp.float32)
    hm = num / (den_rep + 1e-16) + b1_ref[...]
    hm = jnp.maximum(hm, 0.0)
    l2 = jnp.dot(hm, wl_ref[...], preferred_element_type=jnp.float32)
    b = hm.shape[0]
    xr2_ref[...] = jnp.concatenate(
        [jnp.dot(hm, wr_ref[...], preferred_element_type=jnp.float32),
         jnp.zeros((b, 1), jnp.float32)], axis=1)
    xl2_ref[...] = jnp.concatenate(
        [l2, jnp.zeros((b, 1), jnp.float32)], axis=1)


def _out_body(a0_ref, a1_ref, xl_ref, xr_ref, att_ref, b2_ref, o_ref):
    s = a0_ref[...] + a1_ref[...]
    xl = xl_ref[:, :16]
    zz = xl + xr_ref[:, :16]
    m = jnp.maximum(zz, 0.2 * zz) * att_ref[...]
    ee = jnp.exp(jnp.sum(m, axis=1, keepdims=True))
    num = s[:, :16] + ee * xl
    den = s[:, 16:17] + ee
    z = num / (den + 1e-16) + b2_ref[...]
    m = jnp.max(z, axis=1, keepdims=True)
    z = z - m
    lse = jnp.log(jnp.sum(jnp.exp(z), axis=1, keepdims=True))
    o_ref[...] = z - lse


def _full_spec(shape):
    return pl.BlockSpec(shape, lambda i: tuple(0 for _ in shape))


def _row_spec(width):
    return pl.BlockSpec((_ROWBLK, width), lambda i: (i, 0))


def kernel(x, edge_index, W1l, W1r, a1, b1, W2l, W2r, a2, b2):
    n = x.shape[0]
    e0 = edge_index.shape[1]
    pad = EPAD - e0
    src = jnp.concatenate([edge_index[0], jnp.zeros((pad,), jnp.int32)])
    dst = jnp.concatenate([edge_index[1], jnp.full((pad,), DUMMY, jnp.int32)])
    src = src.reshape(NW, NCH, CH)
    dst = dst.reshape(NW, NCH, CH)

    grid = (n // _ROWBLK,)
    xl1, xr1 = pl.pallas_call(
        _mm1_body,
        grid=grid,
        in_specs=[_row_spec(D_FEAT), _full_spec((D_FEAT, 64)),
                  _full_spec((D_FEAT, 64))],
        out_specs=[_row_spec(65), _row_spec(65)],
        out_shape=[jax.ShapeDtypeStruct((n, 65), jnp.float32),
                   jax.ShapeDtypeStruct((n, 65), jnp.float32)],
    )(x, W1l, W1r)

    att1 = jnp.broadcast_to(a1.reshape(-1, 1), (64, 16))
    acc1 = _edge_kernel(HID_HEADS, HID_DIM, 73, 65, 65)(
        src, dst, xl1, xr1, att1)

    xl2, xr2 = pl.pallas_call(
        _mid_body,
        grid=grid,
        in_specs=[_row_spec(73), _row_spec(73), _row_spec(65), _row_spec(65),
                  _full_spec((1, 64)), _full_spec((1, 64)),
                  _full_spec((64, NUM_CLASSES)), _full_spec((64, NUM_CLASSES))],
        out_specs=[_row_spec(17), _row_spec(17)],
        out_shape=[jax.ShapeDtypeStruct((n, 17), jnp.float32),
                   jax.ShapeDtypeStruct((n, 17), jnp.float32)],
    )(acc1[0, :n], acc1[1, :n], xl1, xr1, a1.reshape(1, -1),
      b1.reshape(1, -1), W2l, W2r)

    att2 = jnp.broadcast_to(a2.reshape(-1, 1), (NUM_CLASSES, 16))
    acc2 = _edge_kernel(1, NUM_CLASSES, 17, 17, 17)(
        src, dst, xl2, xr2, att2)

    out = pl.pallas_call(
        _out_body,
        grid=grid,
        in_specs=[_row_spec(17), _row_spec(17), _row_spec(17), _row_spec(17),
                  _full_spec((1, NUM_CLASSES)), _full_spec((1, NUM_CLASSES))],
        out_specs=_row_spec(NUM_CLASSES),
        out_shape=jax.ShapeDtypeStruct((n, NUM_CLASSES), jnp.float32),
    )(acc2[0, :n], acc2[1, :n], xl2, xr2, a2.reshape(1, -1),
      b2.reshape(1, -1))
    return out


# speedup vs baseline: 1.1455x; 1.1455x over previous
import functools

import jax
import jax.numpy as jnp
from jax import lax
from jax.experimental import pallas as pl
from jax.experimental.pallas import tpu as pltpu
from jax.experimental.pallas import tpu_sc as plsc

N_NODES = 10000
N_EDGES = 320000
D_FEAT = 128
HID_HEADS = 8
HID_DIM = 8
NUM_CLASSES = 16

NC = 2
NS = 16
NW = NC * NS
CH = 128
NCH = 82
EPT = NCH * CH
EPAD = EPT * NW
R = 10240
DUMMY = 10016

_ROWBLK = 2000


def _edge_kernel(H, C, WID, XLW, XRW):
    D = H * C
    mesh = plsc.VectorSubcoreMesh(
        core_axis_name="c", subcore_axis_name="s", num_cores=NC,
        num_subcores=NS)
    rows_per_tile = R // NS
    copies = rows_per_tile // CH

    @functools.partial(
        pl.kernel,
        out_type=jax.ShapeDtypeStruct((NC, R, WID), jnp.float32),
        mesh=mesh,
        scratch_types=[
            pltpu.VMEM((NCH, CH), jnp.int32),
            pltpu.VMEM((NCH, CH), jnp.int32),
            pltpu.VMEM((CH, XLW), jnp.float32),
            pltpu.VMEM((CH, XLW), jnp.float32),
            pltpu.VMEM((CH, XRW), jnp.float32),
            pltpu.VMEM((CH, XRW), jnp.float32),
            pltpu.VMEM((CH, WID), jnp.float32),
            pltpu.VMEM((CH, WID), jnp.float32),
            pltpu.VMEM((D, 16), jnp.float32),
            pltpu.VMEM_SHARED((R, WID), jnp.float32),
            pltpu.SemaphoreType.DMA,
            pltpu.SemaphoreType.DMA,
            pltpu.SemaphoreType.DMA,
            pltpu.SemaphoreType.DMA,
            pltpu.SemaphoreType.DMA,
            pltpu.SemaphoreType.DMA,
        ],
        compiler_params=pltpu.CompilerParams(
            needs_layout_passes=False, use_tc_tiling_on_sc=False),
    )
    def ek(src_hbm, dst_hbm, xl_hbm, xr_hbm, att_hbm, out_hbm,
           src_all, dst_all, xl0, xl1, xr0, xr1, w0, w1, att_v, acc_sh,
           gl0, gl1, gr0, gr1, sc0, sc1):
        xls = (xl0, xl1)
        xrs = (xr0, xr1)
        ws = (w0, w1)
        gls = (gl0, gl1)
        grs = (gr0, gr1)
        scs = (sc0, sc1)
        cid = lax.axis_index("c")
        sid = lax.axis_index("s")
        wid = sid * NC + cid
        iota = lax.iota(jnp.int32, 16)
        d_att = pltpu.async_copy(att_hbm, att_v, sc0)
        d_src = pltpu.async_copy(src_hbm.at[wid], src_all, gl0)
        d_dst = pltpu.async_copy(dst_hbm.at[wid], dst_all, gr0)

        for w_v in ws:
            offs = list(range(0, WID - 15, 16))
            if WID % 16:
                offs.append(WID - 16)

            @pl.loop(0, CH)
            def _zw(r, w_v=w_v):
                for off in offs:
                    w_v[r, pl.ds(off, 16)] = jnp.zeros((16,), jnp.float32)

        r0 = sid * rows_per_tile
        dz = [pltpu.async_copy(w0, acc_sh.at[pl.ds(r0 + k * CH, CH)], sc1)
              for k in range(copies)]
        d_att.wait()
        d_src.wait()
        d_dst.wait()
        for d in dz:
            d.wait()
        plsc.subcore_barrier()

        def issue_gather(i, b):
            pltpu.async_copy(xl_hbm.at[src_all.at[i]], xls[b], gls[b])
            pltpu.async_copy(xr_hbm.at[dst_all.at[i]], xrs[b], grs[b])

        def wait_gather(i, b):
            pltpu.make_async_copy(
                xl_hbm.at[src_all.at[i]], xls[b], gls[b]).wait()
            pltpu.make_async_copy(
                xr_hbm.at[dst_all.at[i]], xrs[b], grs[b]).wait()

        def issue_scatter(i, b):
            pltpu.async_copy(ws[b], acc_sh.at[dst_all.at[i]], scs[b],
                             add=True)

        def wait_scatter(i, b):
            pltpu.make_async_copy(
                ws[b], acc_sh.at[dst_all.at[i]], scs[b]).wait()

        def compute(b):
            xl_r, xr_r, w_v = xls[b], xrs[b], ws[b]
            for h in range(H):
                atts = [att_v[h * C + c, :] for c in range(C)]

                @pl.loop(0, CH // 16)
                def _grp(g, h=h, atts=atts):
                    row = g * 16 + iota
                    acc = jnp.zeros((16,), jnp.float32)
                    vls = []
                    for c in range(C):
                        d = h * C + c
                        colv = jnp.full((16,), d, jnp.int32)
                        vl = plsc.load_gather(xl_r, [row, colv])
                        vr = plsc.load_gather(xr_r, [row, colv])
                        z = vl + vr
                        m = jnp.maximum(z, 0.2 * z)
                        acc = acc + m * atts[c]
                        vls.append(vl)
                    ee = jnp.exp(acc)
                    for c in range(C):
                        colv = jnp.full((16,), h * C + c, jnp.int32)
                        plsc.store_scatter(w_v, [row, colv], ee * vls[c])
                    plsc.store_scatter(
                        w_v, [row, jnp.full((16,), D + h, jnp.int32)], ee)

        issue_gather(0, 0)
        issue_gather(1, 1)
        for i in (0, 1):
            wait_gather(i, i)
            compute(i)
            issue_scatter(i, i)
            issue_gather(i + 2, i)

        @pl.loop(1, NCH // 2 - 1)
        def _j(j):
            for b in range(2):
                i = 2 * j + b
                wait_gather(i, b)
                wait_scatter(i - 2, b)
                compute(b)
                issue_scatter(i, b)
                issue_gather(i + 2, b)

        for b in range(2):
            i = NCH - 2 + b
            wait_gather(i, b)
            wait_scatter(i - 2, b)
            compute(b)
            issue_scatter(i, b)
        for b in range(2):
            wait_scatter(NCH - 2 + b, b)

        plsc.subcore_barrier()
        douts = []
        for k in range(copies):
            if k >= 2:
                douts[k - 2].wait()
            pltpu.sync_copy(acc_sh.at[pl.ds(r0 + k * CH, CH)], ws[k % 2])
            douts.append(pltpu.async_copy(
                ws[k % 2], out_hbm.at[cid, pl.ds(r0 + k * CH, CH)],
                scs[k % 2]))
        douts[-2].wait()
        douts[-1].wait()

    return ek


def _mm1_body(x_ref, wl_ref, wr_ref, xl_ref, xr_ref):
    xx = x_ref[...]
    l = jnp.dot(xx, wl_ref[...], preferred_element_type=jnp.float32)
    b = xx.shape[0]
    xr_ref[...] = jnp.concatenate(
        [jnp.dot(xx, wr_ref[...], preferred_element_type=jnp.float32),
         jnp.zeros((b, 1), jnp.float32)], axis=1)
    xl_ref[...] = jnp.concatenate(
        [l, jnp.zeros((b, 1), jnp.float32)], axis=1)


def _mid_body(a0_ref, a1_ref, b1_ref, wl_ref, wr_ref, xl2_ref, xr2_ref):
    s = a0_ref[...] + a1_ref[...]
    num = s[:, :64]
    den = s[:, 64:72]
    rep = (lax.broadcasted_iota(jnp.int32, (8, 64), 1) // 8
           == lax.broadcasted_iota(jnp.int32, (8, 64), 0)).astype(jnp.float32)
    den_rep = jnp.dot(den, rep, preferred_element_type=jnp.float32)
    hm = num / (den_rep + 1e-16) + b1_ref[...]
    hm = jnp.maximum(hm, 0.0)
    l2 = jnp.dot(hm, wl_ref[...], preferred_element_type=jnp.float32)
    b = hm.shape[0]
    xr2_ref[...] = jnp.concatenate(
        [jnp.dot(hm, wr_ref[...], preferred_element_type=jnp.float32),
         jnp.zeros((b, 1), jnp.float32)], axis=1)
    xl2_ref[...] = jnp.concatenate(
        [l2, jnp.zeros((b, 1), jnp.float32)], axis=1)


def _out_body(a0_ref, a1_ref, b2_ref, o_ref):
    s = a0_ref[...] + a1_ref[...]
    num = s[:, :16]
    den = s[:, 16:17]
    z = num / (den + 1e-16) + b2_ref[...]
    m = jnp.max(z, axis=1, keepdims=True)
    z = z - m
    lse = jnp.log(jnp.sum(jnp.exp(z), axis=1, keepdims=True))
    o_ref[...] = z - lse


def _full_spec(shape):
    return pl.BlockSpec(shape, lambda i: tuple(0 for _ in shape))


def _row_spec(width):
    return pl.BlockSpec((_ROWBLK, width), lambda i: (i, 0))


def kernel(x, edge_index, W1l, W1r, a1, b1, W2l, W2r, a2, b2):
    n = x.shape[0]
    e0 = edge_index.shape[1]
    loops = jnp.arange(n, dtype=jnp.int32)
    pad = EPAD - (e0 + n)
    src = jnp.concatenate(
        [edge_index[0], loops, jnp.zeros((pad,), jnp.int32)])
    dst = jnp.concatenate(
        [edge_index[1], loops, jnp.full((pad,), DUMMY, jnp.int32)])
    src = src.reshape(NW, NCH, CH)
    dst = dst.reshape(NW, NCH, CH)

    grid = (n // _ROWBLK,)
    xl1, xr1 = pl.pallas_call(
        _mm1_body,
        grid=grid,
        in_specs=[_row_spec(D_FEAT), _full_spec((D_FEAT, 64)),
                  _full_spec((D_FEAT, 64))],
        out_specs=[_row_spec(65), _row_spec(65)],
        out_shape=[jax.ShapeDtypeStruct((n, 65), jnp.float32),
                   jax.ShapeDtypeStruct((n, 65), jnp.float32)],
    )(x, W1l, W1r)

    att1 = jnp.broadcast_to(a1.reshape(-1, 1), (64, 16))
    acc1 = _edge_kernel(HID_HEADS, HID_DIM, 73, 65, 65)(
        src, dst, xl1, xr1, att1)

    xl2, xr2 = pl.pallas_call(
        _mid_body,
        grid=grid,
        in_specs=[_row_spec(73), _row_spec(73), _full_spec((1, 64)),
                  _full_spec((64, NUM_CLASSES)), _full_spec((64, NUM_CLASSES))],
        out_specs=[_row_spec(17), _row_spec(17)],
        out_shape=[jax.ShapeDtypeStruct((n, 17), jnp.float32),
                   jax.ShapeDtypeStruct((n, 17), jnp.float32)],
    )(acc1[0, :n], acc1[1, :n], b1.reshape(1, -1), W2l, W2r)

    att2 = jnp.broadcast_to(a2.reshape(-1, 1), (NUM_CLASSES, 16))
    acc2 = _edge_kernel(1, NUM_CLASSES, 17, 17, 17)(
        src, dst, xl2, xr2, att2)

    out = pl.pallas_call(
        _out_body,
        grid=grid,
        in_specs=[_row_spec(17), _row_spec(17), _full_spec((1, NUM_CLASSES))],
        out_specs=_row_spec(NUM_CLASSES),
        out_shape=jax.ShapeDtypeStruct((n, NUM_CLASSES), jnp.float32),
    )(acc2[0, :n], acc2[1, :n], b2.reshape(1, -1))
    return out
